# baseline (device time: 188526 ns/iter reference)
import jax
import jax.numpy as jnp
from jax import lax
from jax.experimental import pallas as pl
from jax.experimental.pallas import tpu as pltpu

N_DEV = 4
N_SUB = 2

import os
_ABLATE = int(os.environ.get("KERNEL_ABLATE", "0"))


def kernel(x, w_mat):
    m, k = x.shape
    _, n = w_mat.shape
    mc = m // N_DEV
    ms = mc // N_SUB
    nh = n // 2

    def body(x_ref, w_ref, out_ref, buf_a, buf_b,
             send_a, recv_a, send_b, recv_b):
        my = lax.axis_index("i")
        right = jnp.mod(my + 1, N_DEV)
        left = jnp.mod(my - 1, N_DEV)

        barrier = pltpu.get_barrier_semaphore()
        for nbr in [left, right]:
            pl.semaphore_signal(
                barrier, inc=1,
                device_id=(nbr,), device_id_type=pl.DeviceIdType.MESH,
            )
        pl.semaphore_wait(barrier, 2)

        dirs = [
            dict(dev=right, col=0, buf=buf_a, ssem=send_a, rsem=recv_a, sgn=-1),
            dict(dev=left, col=nh, buf=buf_b, ssem=send_b, rsem=recv_b, sgn=1),
        ]

        def compute_chunk(c):
            sl = pl.ds(c * mc, mc)
            out_ref[sl, :] = jnp.dot(
                x_ref[sl, :].astype(jnp.bfloat16), w_ref[:, :],
                preferred_element_type=jnp.float32,
            ).astype(jnp.bfloat16)

        def rs_desc(D, s, u, chunk):
            return pltpu.make_async_remote_copy(
                src_ref=out_ref.at[pl.ds(chunk * mc + u * ms, ms),
                                   pl.ds(D["col"], nh)],
                dst_ref=D["buf"].at[s, pl.ds(u * ms, ms), :],
                send_sem=D["ssem"].at[N_SUB * s + u],
                recv_sem=D["rsem"].at[N_SUB * s + u],
                device_id=(D["dev"],),
                device_id_type=pl.DeviceIdType.MESH,
            )

        def ag_desc(D, t, u, chunk):
            sl = (pl.ds(chunk * mc + u * ms, ms), pl.ds(D["col"], nh))
            return pltpu.make_async_remote_copy(
                src_ref=out_ref.at[sl],
                dst_ref=out_ref.at[sl],
                send_sem=D["ssem"].at[6 + N_SUB * t + u],
                recv_sem=D["rsem"].at[6 + N_SUB * t + u],
                device_id=(D["dev"],),
                device_id_type=pl.DeviceIdType.MESH,
            )

        c0 = 0.7978845608028654

        def gelu_rows(row_start, n_rows, col, n_cols):
            if _ABLATE == 1:
                return
            for r in range(n_rows // ms):
                sl = (pl.ds(row_start + r * ms, ms), pl.ds(col, n_cols))
                y = out_ref[sl].astype(jnp.float32)
                g = 0.5 * y * (1.0 + jnp.tanh(c0 * (y + 0.044715 * y * y * y)))
                out_ref[sl] = g.astype(jnp.bfloat16)

        if _ABLATE in (1, 3, 5):
            for c in range(N_DEV):
                out_ref[pl.ds(c * mc, mc), :] = jnp.zeros((mc, n), jnp.bfloat16)
        if _ABLATE == 2:
            for c in range(N_DEV):
                compute_chunk(c)
            gelu_rows(0, m, 0, n)
            return
        if _ABLATE == 0:
            compute_chunk(my)
        pend = {}
        for di, D in enumerate(dirs):
            for u in range(N_SUB):
                r = rs_desc(D, 0, u, my)
                r.start()
                pend[(di, 0, u)] = r
        if _ABLATE == 0:
            compute_chunk(jnp.mod(my + 1, N_DEV))
            compute_chunk(jnp.mod(my - 1, N_DEV))
            compute_chunk(jnp.mod(my + 2, N_DEV))

        if _ABLATE == 5:
            for di, D in enumerate(dirs):
                for u in range(N_SUB):
                    pend[(di, 0, u)].wait()
            return

        pend_ag = {}
        for s in range(N_DEV - 1):
            for u in range(N_SUB):
                for di, D in enumerate(dirs):
                    ch = jnp.mod(my + D["sgn"] * (s + 1), N_DEV)
                    pend[(di, s, u)].wait()
                    row = pl.ds(ch * mc + u * ms, ms)
                    col = pl.ds(D["col"], nh)
                    out_ref[row, col] = (
                        out_ref[row, col] + D["buf"][s, pl.ds(u * ms, ms), :]
                    )
                    if s < N_DEV - 2:
                        r = rs_desc(D, s + 1, u, ch)
                        r.start()
                        pend[(di, s + 1, u)] = r
                    elif _ABLATE != 3:
                        own = jnp.mod(my - D["sgn"], N_DEV)
                        r = ag_desc(D, 0, u, own)
                        r.start()
                        pend_ag[(di, 0, u)] = r

        if _ABLATE == 3:
            return

        for t in range(N_DEV - 1):
            for u in range(N_SUB):
                for di, D in enumerate(dirs):
                    pend_ag[(di, t, u)].wait()
                    if t < N_DEV - 2:
                        ch = jnp.mod(my + D["sgn"] * t, N_DEV)
                        r = ag_desc(D, t + 1, u, ch)
                        r.start()
                        pend_ag[(di, t + 1, u)] = r
            if t == 0:
                gelu_rows_a = jnp.mod(my + 1, N_DEV)
                gelu_rows_b = jnp.mod(my - 1, N_DEV)
                gelu_rows(gelu_rows_a * mc, mc, 0, nh)
                gelu_rows(gelu_rows_b * mc, mc, nh, nh)
            elif t == 1:
                gelu_rows(my * mc, mc, 0, n)
        gelu_rows(jnp.mod(my - 1, N_DEV) * mc, mc, 0, nh)
        gelu_rows(jnp.mod(my + 1, N_DEV) * mc, mc, nh, nh)
        gelu_rows(jnp.mod(my + 2, N_DEV) * mc, mc, 0, n)

    n_sem = N_SUB * 2 * (N_DEV - 1)
    return pl.pallas_call(
        body,
        out_shape=jax.ShapeDtypeStruct((m, n), jnp.bfloat16),
        in_specs=[
            pl.BlockSpec(memory_space=pltpu.VMEM),
            pl.BlockSpec(memory_space=pltpu.VMEM),
        ],
        out_specs=pl.BlockSpec(memory_space=pltpu.VMEM),
        scratch_shapes=[
            pltpu.VMEM((N_DEV - 1, mc, nh), jnp.bfloat16),
            pltpu.VMEM((N_DEV - 1, mc, nh), jnp.bfloat16),
            pltpu.SemaphoreType.DMA((n_sem,)),
            pltpu.SemaphoreType.DMA((n_sem,)),
            pltpu.SemaphoreType.DMA((n_sem,)),
            pltpu.SemaphoreType.DMA((n_sem,)),
        ],
        compiler_params=pltpu.CompilerParams(
            collective_id=0,
            vmem_limit_bytes=60 * 1024 * 1024,
        ),
    )(x, w_mat.astype(jnp.bfloat16))


# device time: 174248 ns/iter; 1.0819x vs baseline; 1.0819x over previous
import jax
import jax.numpy as jnp
from jax import lax
from jax.experimental import pallas as pl
from jax.experimental.pallas import tpu as pltpu

N_DEV = 4
N_SUB = 2


def kernel(x, w_mat):
    m, k = x.shape
    _, n = w_mat.shape
    mc = m // N_DEV
    ms = mc // N_SUB
    nh = n // 2

    def body(x_ref, w_ref, out_ref, xbuf, wbf, buf_a, buf_b,
             xsems, send_a, recv_a, send_b, recv_b):
        my = lax.axis_index("i")
        right = jnp.mod(my + 1, N_DEV)
        left = jnp.mod(my - 1, N_DEV)

        def x_load(slot, c):
            cp = pltpu.make_async_copy(
                x_ref.at[pl.ds(c * mc, mc), :], xbuf.at[slot], xsems.at[slot]
            )
            cp.start()
            return cp

        ld = x_load(0, my)
        wbf[:, :] = w_ref[:, :].astype(jnp.bfloat16)

        barrier = pltpu.get_barrier_semaphore()
        for nbr in [left, right]:
            pl.semaphore_signal(
                barrier, inc=1,
                device_id=(nbr,), device_id_type=pl.DeviceIdType.MESH,
            )
        pl.semaphore_wait(barrier, 2)

        dirs = [
            dict(dev=right, col=0, buf=buf_a, ssem=send_a, rsem=recv_a, sgn=-1),
            dict(dev=left, col=nh, buf=buf_b, ssem=send_b, rsem=recv_b, sgn=1),
        ]

        def compute_chunk(c, slot, pending_load):
            pending_load.wait()
            out_ref[pl.ds(c * mc, mc), :] = jnp.dot(
                xbuf[slot].astype(jnp.bfloat16), wbf[:, :],
                preferred_element_type=jnp.float32,
            ).astype(jnp.bfloat16)

        def rs_desc(D, s, u, chunk):
            return pltpu.make_async_remote_copy(
                src_ref=out_ref.at[pl.ds(chunk * mc + u * ms, ms),
                                   pl.ds(D["col"], nh)],
                dst_ref=D["buf"].at[s, pl.ds(u * ms, ms), :],
                send_sem=D["ssem"].at[N_SUB * s + u],
                recv_sem=D["rsem"].at[N_SUB * s + u],
                device_id=(D["dev"],),
                device_id_type=pl.DeviceIdType.MESH,
            )

        def ag_desc(D, t, u, chunk):
            sl = (pl.ds(chunk * mc + u * ms, ms), pl.ds(D["col"], nh))
            return pltpu.make_async_remote_copy(
                src_ref=out_ref.at[sl],
                dst_ref=out_ref.at[sl],
                send_sem=D["ssem"].at[6 + N_SUB * t + u],
                recv_sem=D["rsem"].at[6 + N_SUB * t + u],
                device_id=(D["dev"],),
                device_id_type=pl.DeviceIdType.MESH,
            )

        c0 = 0.7978845608028654

        def gelu_rows(row_start, n_rows, col, n_cols):
            for r in range(n_rows // ms):
                sl = (pl.ds(row_start + r * ms, ms), pl.ds(col, n_cols))
                y = out_ref[sl].astype(jnp.float32)
                g = 0.5 * y * (1.0 + jnp.tanh(c0 * (y + 0.044715 * y * y * y)))
                out_ref[sl] = g.astype(jnp.bfloat16)

        ld2 = x_load(1, jnp.mod(my + 1, N_DEV))
        compute_chunk(my, 0, ld)
        pend = {}
        for di, D in enumerate(dirs):
            for u in range(N_SUB):
                r = rs_desc(D, 0, u, my)
                r.start()
                pend[(di, 0, u)] = r
        ld3 = x_load(0, jnp.mod(my - 1, N_DEV))
        compute_chunk(jnp.mod(my + 1, N_DEV), 1, ld2)
        ld4 = x_load(1, jnp.mod(my + 2, N_DEV))
        compute_chunk(jnp.mod(my - 1, N_DEV), 0, ld3)
        compute_chunk(jnp.mod(my + 2, N_DEV), 1, ld4)

        pend_ag = {}
        for s in range(N_DEV - 1):
            for u in range(N_SUB):
                for di, D in enumerate(dirs):
                    ch = jnp.mod(my + D["sgn"] * (s + 1), N_DEV)
                    pend[(di, s, u)].wait()
                    row = pl.ds(ch * mc + u * ms, ms)
                    col = pl.ds(D["col"], nh)
                    out_ref[row, col] = (
                        out_ref[row, col] + D["buf"][s, pl.ds(u * ms, ms), :]
                    )
                    if s < N_DEV - 2:
                        r = rs_desc(D, s + 1, u, ch)
                        r.start()
                        pend[(di, s + 1, u)] = r
                    else:
                        own = jnp.mod(my - D["sgn"], N_DEV)
                        r = ag_desc(D, 0, u, own)
                        r.start()
                        pend_ag[(di, 0, u)] = r

        for t in range(N_DEV - 2):
            for u in range(N_SUB):
                for di, D in enumerate(dirs):
                    pend_ag[(di, t, u)].wait()
                    ch = jnp.mod(my + D["sgn"] * t, N_DEV)
                    r = ag_desc(D, t + 1, u, ch)
                    r.start()
                    pend_ag[(di, t + 1, u)] = r
            if t == 0:
                gelu_rows(jnp.mod(my + 1, N_DEV) * mc, mc, 0, nh)
                gelu_rows(jnp.mod(my - 1, N_DEV) * mc, mc, nh, nh)
            elif t == 1:
                gelu_rows(my * mc, mc, 0, n)
        for u in range(N_SUB):
            for di, D in enumerate(dirs):
                pend_ag[(di, N_DEV - 2, u)].wait()
            gelu_rows(jnp.mod(my + 2, N_DEV) * mc + u * ms, ms, 0, n)
            gelu_rows(jnp.mod(my - 1, N_DEV) * mc + u * ms, ms, 0, nh)
            gelu_rows(jnp.mod(my + 1, N_DEV) * mc + u * ms, ms, nh, nh)

    n_sem = N_SUB * 2 * (N_DEV - 1)
    return pl.pallas_call(
        body,
        out_shape=jax.ShapeDtypeStruct((m, n), jnp.bfloat16),
        in_specs=[
            pl.BlockSpec(memory_space=pl.ANY),
            pl.BlockSpec(memory_space=pltpu.VMEM),
        ],
        out_specs=pl.BlockSpec(memory_space=pltpu.VMEM),
        scratch_shapes=[
            pltpu.VMEM((2, mc, k), jnp.float32),
            pltpu.VMEM((k, n), jnp.bfloat16),
            pltpu.VMEM((N_DEV - 1, mc, nh), jnp.bfloat16),
            pltpu.VMEM((N_DEV - 1, mc, nh), jnp.bfloat16),
            pltpu.SemaphoreType.DMA((2,)),
            pltpu.SemaphoreType.DMA((n_sem,)),
            pltpu.SemaphoreType.DMA((n_sem,)),
            pltpu.SemaphoreType.DMA((n_sem,)),
            pltpu.SemaphoreType.DMA((n_sem,)),
        ],
        compiler_params=pltpu.CompilerParams(
            collective_id=0,
            vmem_limit_bytes=60 * 1024 * 1024,
        ),
    )(x, w_mat)


# device time: 167316 ns/iter; 1.1268x vs baseline; 1.0414x over previous
import jax
import jax.numpy as jnp
from jax import lax
from jax.experimental import pallas as pl
from jax.experimental.pallas import tpu as pltpu

N_DEV = 4
N_SUB = 2


def kernel(x, w_mat):
    m, k = x.shape
    _, n = w_mat.shape
    mc = m // N_DEV
    ms = mc // N_SUB
    nh = n // 2

    def body(x_ref, w_ref, out_ref, xbuf, wbf, acc, buf_a, buf_b,
             xsems, osems, send_a, recv_a, send_b, recv_b):
        my = lax.axis_index("i")
        right = jnp.mod(my + 1, N_DEV)
        left = jnp.mod(my - 1, N_DEV)

        def x_load(slot, c):
            cp = pltpu.make_async_copy(
                x_ref.at[pl.ds(c * mc, mc), :], xbuf.at[slot], xsems.at[slot]
            )
            cp.start()
            return cp

        ld = x_load(0, my)
        wbf[:, :] = w_ref[:, :].astype(jnp.bfloat16)

        barrier = pltpu.get_barrier_semaphore()
        for nbr in [left, right]:
            pl.semaphore_signal(
                barrier, inc=1,
                device_id=(nbr,), device_id_type=pl.DeviceIdType.MESH,
            )
        pl.semaphore_wait(barrier, 2)

        dirs = [
            dict(dev=right, col=0, buf=buf_a, ssem=send_a, rsem=recv_a, sgn=-1),
            dict(dev=left, col=nh, buf=buf_b, ssem=send_b, rsem=recv_b, sgn=1),
        ]

        def compute_rows(c, slot, u0, n_sub):
            acc[pl.ds(c * mc + u0 * ms, n_sub * ms), :] = jnp.dot(
                xbuf[slot, pl.ds(u0 * ms, n_sub * ms), :].astype(jnp.bfloat16),
                wbf[:, :],
                preferred_element_type=jnp.float32,
            ).astype(jnp.bfloat16)

        def rs_desc(D, s, u, chunk):
            return pltpu.make_async_remote_copy(
                src_ref=acc.at[pl.ds(chunk * mc + u * ms, ms),
                               pl.ds(D["col"], nh)],
                dst_ref=D["buf"].at[s, pl.ds(u * ms, ms), :],
                send_sem=D["ssem"].at[N_SUB * s + u],
                recv_sem=D["rsem"].at[N_SUB * s + u],
                device_id=(D["dev"],),
                device_id_type=pl.DeviceIdType.MESH,
            )

        def ag_desc(D, t, u, chunk):
            sl = (pl.ds(chunk * mc + u * ms, ms), pl.ds(D["col"], nh))
            return pltpu.make_async_remote_copy(
                src_ref=acc.at[sl],
                dst_ref=acc.at[sl],
                send_sem=D["ssem"].at[6 + N_SUB * t + u],
                recv_sem=D["rsem"].at[6 + N_SUB * t + u],
                device_id=(D["dev"],),
                device_id_type=pl.DeviceIdType.MESH,
            )

        c0 = 0.7978845608028654
        out_copies = []

        def gelu_out(row_start, n_rows, col, n_cols):
            for r in range(n_rows // ms):
                sl = (pl.ds(row_start + r * ms, ms), pl.ds(col, n_cols))
                y = acc[sl].astype(jnp.float32)
                g = 0.5 * y * (1.0 + jnp.tanh(c0 * (y + 0.044715 * y * y * y)))
                acc[sl] = g.astype(jnp.bfloat16)
            sl = (pl.ds(row_start, n_rows), pl.ds(col, n_cols))
            cp = pltpu.make_async_copy(
                acc.at[sl], out_ref.at[sl], osems.at[len(out_copies)]
            )
            cp.start()
            out_copies.append(cp)

        ld2 = x_load(1, jnp.mod(my + 1, N_DEV))
        ld.wait()
        pend = {}
        for u in range(N_SUB):
            compute_rows(my, 0, u, 1)
            for di, D in enumerate(dirs):
                r = rs_desc(D, 0, u, my)
                r.start()
                pend[(di, 0, u)] = r
        ld3 = x_load(0, jnp.mod(my - 1, N_DEV))
        ld2.wait()
        compute_rows(jnp.mod(my + 1, N_DEV), 1, 0, N_SUB)
        ld4 = x_load(1, jnp.mod(my + 2, N_DEV))
        ld3.wait()
        compute_rows(jnp.mod(my - 1, N_DEV), 0, 0, N_SUB)
        ld4.wait()
        compute_rows(jnp.mod(my + 2, N_DEV), 1, 0, N_SUB)

        pend_ag = {}
        for s in range(N_DEV - 1):
            for u in range(N_SUB):
                for di, D in enumerate(dirs):
                    ch = jnp.mod(my + D["sgn"] * (s + 1), N_DEV)
                    pend[(di, s, u)].wait()
                    row = pl.ds(ch * mc + u * ms, ms)
                    col = pl.ds(D["col"], nh)
                    acc[row, col] = acc[row, col] + D["buf"][s, pl.ds(u * ms, ms), :]
                    if s < N_DEV - 2:
                        r = rs_desc(D, s + 1, u, ch)
                        r.start()
                        pend[(di, s + 1, u)] = r
                    else:
                        own = jnp.mod(my - D["sgn"], N_DEV)
                        r = ag_desc(D, 0, u, own)
                        r.start()
                        pend_ag[(di, 0, u)] = r

        for t in range(N_DEV - 2):
            for u in range(N_SUB):
                for di, D in enumerate(dirs):
                    pend_ag[(di, t, u)].wait()
                    ch = jnp.mod(my + D["sgn"] * t, N_DEV)
                    r = ag_desc(D, t + 1, u, ch)
                    r.start()
                    pend_ag[(di, t + 1, u)] = r
            if t == 0:
                gelu_out(jnp.mod(my + 1, N_DEV) * mc, mc, 0, nh)
                gelu_out(jnp.mod(my - 1, N_DEV) * mc, mc, nh, nh)
            elif t == 1:
                gelu_out(my * mc, mc, 0, n)
        for u in range(N_SUB):
            for di, D in enumerate(dirs):
                pend_ag[(di, N_DEV - 2, u)].wait()
            gelu_out(jnp.mod(my + 2, N_DEV) * mc + u * ms, ms, 0, n)
            gelu_out(jnp.mod(my - 1, N_DEV) * mc + u * ms, ms, 0, nh)
            gelu_out(jnp.mod(my + 1, N_DEV) * mc + u * ms, ms, nh, nh)

        for cp in out_copies:
            cp.wait()

    n_sem = N_SUB * 2 * (N_DEV - 1)
    return pl.pallas_call(
        body,
        out_shape=jax.ShapeDtypeStruct((m, n), jnp.bfloat16),
        in_specs=[
            pl.BlockSpec(memory_space=pl.ANY),
            pl.BlockSpec(memory_space=pltpu.VMEM),
        ],
        out_specs=pl.BlockSpec(memory_space=pl.ANY),
        scratch_shapes=[
            pltpu.VMEM((2, mc, k), jnp.float32),
            pltpu.VMEM((k, n), jnp.bfloat16),
            pltpu.VMEM((m, n), jnp.bfloat16),
            pltpu.VMEM((N_DEV - 1, mc, nh), jnp.bfloat16),
            pltpu.VMEM((N_DEV - 1, mc, nh), jnp.bfloat16),
            pltpu.SemaphoreType.DMA((2,)),
            pltpu.SemaphoreType.DMA((9,)),
            pltpu.SemaphoreType.DMA((n_sem,)),
            pltpu.SemaphoreType.DMA((n_sem,)),
            pltpu.SemaphoreType.DMA((n_sem,)),
            pltpu.SemaphoreType.DMA((n_sem,)),
        ],
        compiler_params=pltpu.CompilerParams(
            collective_id=0,
            vmem_limit_bytes=60 * 1024 * 1024,
        ),
    )(x, w_mat)


# device time: 163900 ns/iter; 1.1503x vs baseline; 1.0208x over previous
import jax
import jax.numpy as jnp
from jax import lax
from jax.experimental import pallas as pl
from jax.experimental.pallas import tpu as pltpu

N_DEV = 4
N_SUB = 4


def kernel(x, w_mat):
    m, k = x.shape
    _, n = w_mat.shape
    mc = m // N_DEV
    ms = mc // N_SUB
    nh = n // 2

    def body(x_ref, w_ref, out_ref, xbuf, wbf, acc, buf_a, buf_b,
             xsems, osems, send_a, recv_a, send_b, recv_b):
        my = lax.axis_index("i")
        right = jnp.mod(my + 1, N_DEV)
        left = jnp.mod(my - 1, N_DEV)

        def x_load(slot, c):
            cp = pltpu.make_async_copy(
                x_ref.at[pl.ds(c * mc, mc), :], xbuf.at[slot], xsems.at[slot]
            )
            cp.start()
            return cp

        ld = x_load(0, my)
        wbf[:, :] = w_ref[:, :].astype(jnp.bfloat16)

        barrier = pltpu.get_barrier_semaphore()
        for nbr in [left, right]:
            pl.semaphore_signal(
                barrier, inc=1,
                device_id=(nbr,), device_id_type=pl.DeviceIdType.MESH,
            )
        pl.semaphore_wait(barrier, 2)

        dirs = [
            dict(dev=right, col=0, buf=buf_a, ssem=send_a, rsem=recv_a, sgn=-1),
            dict(dev=left, col=nh, buf=buf_b, ssem=send_b, rsem=recv_b, sgn=1),
        ]

        def compute_rows(c, slot, u0, n_sub):
            acc[pl.ds(c * mc + u0 * ms, n_sub * ms), :] = jnp.dot(
                xbuf[slot, pl.ds(u0 * ms, n_sub * ms), :].astype(jnp.bfloat16),
                wbf[:, :],
                preferred_element_type=jnp.float32,
            ).astype(jnp.bfloat16)

        def rs_desc(D, s, u, chunk):
            return pltpu.make_async_remote_copy(
                src_ref=acc.at[pl.ds(chunk * mc + u * ms, ms),
                               pl.ds(D["col"], nh)],
                dst_ref=D["buf"].at[s, pl.ds(u * ms, ms), :],
                send_sem=D["ssem"].at[N_SUB * s + u],
                recv_sem=D["rsem"].at[N_SUB * s + u],
                device_id=(D["dev"],),
                device_id_type=pl.DeviceIdType.MESH,
            )

        def ag_desc(D, t, u, chunk):
            sl = (pl.ds(chunk * mc + u * ms, ms), pl.ds(D["col"], nh))
            return pltpu.make_async_remote_copy(
                src_ref=acc.at[sl],
                dst_ref=acc.at[sl],
                send_sem=D["ssem"].at[N_SUB * (N_DEV - 1) + N_SUB * t + u],
                recv_sem=D["rsem"].at[N_SUB * (N_DEV - 1) + N_SUB * t + u],
                device_id=(D["dev"],),
                device_id_type=pl.DeviceIdType.MESH,
            )

        c0 = 0.7978845608028654
        out_copies = []

        def gelu_out(row_start, n_rows, col, n_cols):
            for r in range(n_rows // ms):
                sl = (pl.ds(row_start + r * ms, ms), pl.ds(col, n_cols))
                y = acc[sl].astype(jnp.float32)
                g = 0.5 * y * (1.0 + jnp.tanh(c0 * (y + 0.044715 * y * y * y)))
                acc[sl] = g.astype(jnp.bfloat16)
            sl = (pl.ds(row_start, n_rows), pl.ds(col, n_cols))
            cp = pltpu.make_async_copy(
                acc.at[sl], out_ref.at[sl], osems.at[len(out_copies)]
            )
            cp.start()
            out_copies.append(cp)

        ld2 = x_load(1, jnp.mod(my + 1, N_DEV))
        ld.wait()
        pend = {}
        for u in range(N_SUB):
            compute_rows(my, 0, u, 1)
            for di, D in enumerate(dirs):
                r = rs_desc(D, 0, u, my)
                r.start()
                pend[(di, 0, u)] = r
        ld3 = x_load(0, jnp.mod(my - 1, N_DEV))
        ld2.wait()
        compute_rows(jnp.mod(my + 1, N_DEV), 1, 0, N_SUB)
        ld4 = x_load(1, jnp.mod(my + 2, N_DEV))
        ld3.wait()
        compute_rows(jnp.mod(my - 1, N_DEV), 0, 0, N_SUB)
        ld4.wait()
        compute_rows(jnp.mod(my + 2, N_DEV), 1, 0, N_SUB)

        pend_ag = {}
        for s in range(N_DEV - 1):
            for u in range(N_SUB):
                for di, D in enumerate(dirs):
                    ch = jnp.mod(my + D["sgn"] * (s + 1), N_DEV)
                    pend[(di, s, u)].wait()
                    row = pl.ds(ch * mc + u * ms, ms)
                    col = pl.ds(D["col"], nh)
                    acc[row, col] = acc[row, col] + D["buf"][s, pl.ds(u * ms, ms), :]
                    if s < N_DEV - 2:
                        r = rs_desc(D, s + 1, u, ch)
                        r.start()
                        pend[(di, s + 1, u)] = r
                    else:
                        own = jnp.mod(my - D["sgn"], N_DEV)
                        r = ag_desc(D, 0, u, own)
                        r.start()
                        pend_ag[(di, 0, u)] = r

        for t in range(N_DEV - 2):
            for u in range(N_SUB):
                for di, D in enumerate(dirs):
                    pend_ag[(di, t, u)].wait()
                    ch = jnp.mod(my + D["sgn"] * t, N_DEV)
                    r = ag_desc(D, t + 1, u, ch)
                    r.start()
                    pend_ag[(di, t + 1, u)] = r
            if t == 0:
                gelu_out(jnp.mod(my + 1, N_DEV) * mc, mc, 0, nh)
                gelu_out(jnp.mod(my - 1, N_DEV) * mc, mc, nh, nh)
            elif t == 1:
                gelu_out(my * mc, mc, 0, n)
        for u in range(N_SUB):
            for di, D in enumerate(dirs):
                pend_ag[(di, N_DEV - 2, u)].wait()
            gelu_out(jnp.mod(my + 2, N_DEV) * mc + u * ms, ms, 0, n)
            gelu_out(jnp.mod(my - 1, N_DEV) * mc + u * ms, ms, 0, nh)
            gelu_out(jnp.mod(my + 1, N_DEV) * mc + u * ms, ms, nh, nh)

        for cp in out_copies:
            cp.wait()

    n_sem = N_SUB * 2 * (N_DEV - 1)
    return pl.pallas_call(
        body,
        out_shape=jax.ShapeDtypeStruct((m, n), jnp.bfloat16),
        in_specs=[
            pl.BlockSpec(memory_space=pl.ANY),
            pl.BlockSpec(memory_space=pltpu.VMEM),
        ],
        out_specs=pl.BlockSpec(memory_space=pl.ANY),
        scratch_shapes=[
            pltpu.VMEM((2, mc, k), jnp.float32),
            pltpu.VMEM((k, n), jnp.bfloat16),
            pltpu.VMEM((m, n), jnp.bfloat16),
            pltpu.VMEM((N_DEV - 1, mc, nh), jnp.bfloat16),
            pltpu.VMEM((N_DEV - 1, mc, nh), jnp.bfloat16),
            pltpu.SemaphoreType.DMA((2,)),
            pltpu.SemaphoreType.DMA((3 + 3 * N_SUB,)),
            pltpu.SemaphoreType.DMA((n_sem,)),
            pltpu.SemaphoreType.DMA((n_sem,)),
            pltpu.SemaphoreType.DMA((n_sem,)),
            pltpu.SemaphoreType.DMA((n_sem,)),
        ],
        compiler_params=pltpu.CompilerParams(
            collective_id=0,
            vmem_limit_bytes=60 * 1024 * 1024,
        ),
    )(x, w_mat)


# device time: 163009 ns/iter; 1.1565x vs baseline; 1.0055x over previous
import jax
import jax.numpy as jnp
from jax import lax
from jax.experimental import pallas as pl
from jax.experimental.pallas import tpu as pltpu

N_DEV = 4
N_SUB = 8


def kernel(x, w_mat):
    m, k = x.shape
    _, n = w_mat.shape
    mc = m // N_DEV
    ms = mc // N_SUB
    nh = n // 2

    def body(x_ref, w_ref, out_ref, xbuf, wbf, acc, buf_a, buf_b,
             xsems, osems, send_a, recv_a, send_b, recv_b):
        my = lax.axis_index("i")
        right = jnp.mod(my + 1, N_DEV)
        left = jnp.mod(my - 1, N_DEV)

        def x_load(slot, c):
            cp = pltpu.make_async_copy(
                x_ref.at[pl.ds(c * mc, mc), :], xbuf.at[slot], xsems.at[slot]
            )
            cp.start()
            return cp

        ld = x_load(0, my)
        wbf[:, :] = w_ref[:, :].astype(jnp.bfloat16)

        barrier = pltpu.get_barrier_semaphore()
        for nbr in [left, right]:
            pl.semaphore_signal(
                barrier, inc=1,
                device_id=(nbr,), device_id_type=pl.DeviceIdType.MESH,
            )
        pl.semaphore_wait(barrier, 2)

        dirs = [
            dict(dev=right, col=0, buf=buf_a, ssem=send_a, rsem=recv_a, sgn=-1),
            dict(dev=left, col=nh, buf=buf_b, ssem=send_b, rsem=recv_b, sgn=1),
        ]

        def compute_rows(c, slot, u0, n_sub):
            acc[pl.ds(c * mc + u0 * ms, n_sub * ms), :] = jnp.dot(
                xbuf[slot, pl.ds(u0 * ms, n_sub * ms), :].astype(jnp.bfloat16),
                wbf[:, :],
                preferred_element_type=jnp.float32,
            ).astype(jnp.bfloat16)

        def rs_desc(D, s, u, chunk):
            return pltpu.make_async_remote_copy(
                src_ref=acc.at[pl.ds(chunk * mc + u * ms, ms),
                               pl.ds(D["col"], nh)],
                dst_ref=D["buf"].at[s, pl.ds(u * ms, ms), :],
                send_sem=D["ssem"].at[N_SUB * s + u],
                recv_sem=D["rsem"].at[N_SUB * s + u],
                device_id=(D["dev"],),
                device_id_type=pl.DeviceIdType.MESH,
            )

        def ag_desc(D, t, u, chunk):
            sl = (pl.ds(chunk * mc + u * ms, ms), pl.ds(D["col"], nh))
            return pltpu.make_async_remote_copy(
                src_ref=acc.at[sl],
                dst_ref=acc.at[sl],
                send_sem=D["ssem"].at[N_SUB * (N_DEV - 1) + N_SUB * t + u],
                recv_sem=D["rsem"].at[N_SUB * (N_DEV - 1) + N_SUB * t + u],
                device_id=(D["dev"],),
                device_id_type=pl.DeviceIdType.MESH,
            )

        c0 = 0.7978845608028654
        out_copies = []

        def gelu_out(row_start, n_rows, col, n_cols):
            for r in range(n_rows // ms):
                sl = (pl.ds(row_start + r * ms, ms), pl.ds(col, n_cols))
                y = acc[sl].astype(jnp.float32)
                g = 0.5 * y * (1.0 + jnp.tanh(c0 * (y + 0.044715 * y * y * y)))
                acc[sl] = g.astype(jnp.bfloat16)
            sl = (pl.ds(row_start, n_rows), pl.ds(col, n_cols))
            cp = pltpu.make_async_copy(
                acc.at[sl], out_ref.at[sl], osems.at[len(out_copies)]
            )
            cp.start()
            out_copies.append(cp)

        ld2 = x_load(1, jnp.mod(my + 1, N_DEV))
        ld.wait()
        pend = {}
        for u in range(N_SUB):
            compute_rows(my, 0, u, 1)
            for di, D in enumerate(dirs):
                r = rs_desc(D, 0, u, my)
                r.start()
                pend[(di, 0, u)] = r
        ld3 = x_load(0, jnp.mod(my - 1, N_DEV))
        ld2.wait()
        compute_rows(jnp.mod(my + 1, N_DEV), 1, 0, N_SUB)
        ld4 = x_load(1, jnp.mod(my + 2, N_DEV))
        ld3.wait()
        compute_rows(jnp.mod(my - 1, N_DEV), 0, 0, N_SUB)
        ld4.wait()
        compute_rows(jnp.mod(my + 2, N_DEV), 1, 0, N_SUB)

        pend_ag = {}
        for s in range(N_DEV - 1):
            for u in range(N_SUB):
                for di, D in enumerate(dirs):
                    ch = jnp.mod(my + D["sgn"] * (s + 1), N_DEV)
                    pend[(di, s, u)].wait()
                    row = pl.ds(ch * mc + u * ms, ms)
                    col = pl.ds(D["col"], nh)
                    acc[row, col] = acc[row, col] + D["buf"][s, pl.ds(u * ms, ms), :]
                    if s < N_DEV - 2:
                        r = rs_desc(D, s + 1, u, ch)
                        r.start()
                        pend[(di, s + 1, u)] = r
                    else:
                        own = jnp.mod(my - D["sgn"], N_DEV)
                        r = ag_desc(D, 0, u, own)
                        r.start()
                        pend_ag[(di, 0, u)] = r

        for t in range(N_DEV - 2):
            for u in range(N_SUB):
                for di, D in enumerate(dirs):
                    pend_ag[(di, t, u)].wait()
                    ch = jnp.mod(my + D["sgn"] * t, N_DEV)
                    r = ag_desc(D, t + 1, u, ch)
                    r.start()
                    pend_ag[(di, t + 1, u)] = r
            if t == 0:
                gelu_out(jnp.mod(my + 1, N_DEV) * mc, mc, 0, nh)
                gelu_out(jnp.mod(my - 1, N_DEV) * mc, mc, nh, nh)
            elif t == 1:
                gelu_out(my * mc, mc, 0, n)
        for u in range(N_SUB):
            for di, D in enumerate(dirs):
                pend_ag[(di, N_DEV - 2, u)].wait()
            gelu_out(jnp.mod(my + 2, N_DEV) * mc + u * ms, ms, 0, n)
            gelu_out(jnp.mod(my - 1, N_DEV) * mc + u * ms, ms, 0, nh)
            gelu_out(jnp.mod(my + 1, N_DEV) * mc + u * ms, ms, nh, nh)

        for cp in out_copies:
            cp.wait()

    n_sem = N_SUB * 2 * (N_DEV - 1)
    return pl.pallas_call(
        body,
        out_shape=jax.ShapeDtypeStruct((m, n), jnp.bfloat16),
        in_specs=[
            pl.BlockSpec(memory_space=pl.ANY),
            pl.BlockSpec(memory_space=pltpu.VMEM),
        ],
        out_specs=pl.BlockSpec(memory_space=pl.ANY),
        scratch_shapes=[
            pltpu.VMEM((2, mc, k), jnp.float32),
            pltpu.VMEM((k, n), jnp.bfloat16),
            pltpu.VMEM((m, n), jnp.bfloat16),
            pltpu.VMEM((N_DEV - 1, mc, nh), jnp.bfloat16),
            pltpu.VMEM((N_DEV - 1, mc, nh), jnp.bfloat16),
            pltpu.SemaphoreType.DMA((2,)),
            pltpu.SemaphoreType.DMA((3 + 3 * N_SUB,)),
            pltpu.SemaphoreType.DMA((n_sem,)),
            pltpu.SemaphoreType.DMA((n_sem,)),
            pltpu.SemaphoreType.DMA((n_sem,)),
            pltpu.SemaphoreType.DMA((n_sem,)),
        ],
        compiler_params=pltpu.CompilerParams(
            collective_id=0,
            vmem_limit_bytes=60 * 1024 * 1024,
        ),
    )(x, w_mat)
